# Initial kernel scaffold; baseline (speedup 1.0000x reference)
import jax
import jax.numpy as jnp
from jax.experimental import pallas as pl


def kernel(inputs):
    raise NotImplementedError("write your pallas kernel here")


# speedup vs baseline: 31.6483x; 31.6483x over previous
import functools

import numpy as np
import jax
import jax.numpy as jnp
from jax import lax
from jax.experimental import pallas as pl
from jax.experimental.pallas import tpu as pltpu
from jax.experimental.pallas import tpu_sc as plsc

B = 16
T = 1024
N = T - 1
D = T + 1
DP = 1040
R = 16
NCHUNK = 512 // R
W = 2064
NBLK = DP // 16
GQ = 5
GB = NBLK // GQ


def _inv_counts() -> np.ndarray:
    si = np.arange(DP)
    cnt = N - np.abs(si - (D // 2))
    return np.where(si < D, 1.0 / np.maximum(cnt, 1), 0.0).astype(np.float32)


_INVC = _inv_counts()

_mesh = plsc.VectorSubcoreMesh(core_axis_name="c", subcore_axis_name="s")


@functools.partial(
    pl.kernel,
    out_type=jax.ShapeDtypeStruct((B, DP), jnp.float32),
    mesh=_mesh,
    scratch_types=[
        pltpu.VMEM((R, W), jnp.float32),
        pltpu.VMEM((DP,), jnp.float32),
        pltpu.VMEM((DP,), jnp.float32),
        pltpu.VMEM((DP,), jnp.float32),
        pltpu.VMEM((DP,), jnp.float32),
        pltpu.VMEM_SHARED((16, DP), jnp.float32),
    ],
    compiler_params=pltpu.CompilerParams(
        use_tc_tiling_on_sc=False, needs_layout_passes=False),
)
def _diag_mean_sc(in_hbm, invc_hbm, out_hbm, rows_ref, acc_ref, tmp_ref,
                  invc_ref, obuf_ref, shared_ref):
    _ZERO16 = jnp.zeros((16,), jnp.float32)
    lastcol_mask = jnp.where(
        lax.iota(jnp.int32, 16) < 15, jnp.float32(1.0), jnp.float32(0.0))
    c = lax.axis_index("c")
    s = lax.axis_index("s")
    half = s % 2
    b = c * 8 + s // 2

    pltpu.sync_copy(invc_hbm, invc_ref)

    def zero_row(r, carry):
        for k in range(W // 16):
            rows_ref[r, pl.ds(16 * k, 16)] = _ZERO16
        return carry

    lax.fori_loop(0, R, zero_row, 0)

    for k in range(NBLK):
        acc_ref[pl.ds(16 * k, 16)] = _ZERO16

    y_base = half * 512

    def chunk_body(ch, carry):
        y0 = y_base + ch * R
        pltpu.sync_copy(
            in_hbm.at[b, pl.ds(y0, R), :],
            rows_ref.at[:, pl.ds(512, T)],
        )

        def mask_last(r, carry):
            rows_ref[r, pl.ds(1520, 16)] = (
                rows_ref[r, pl.ds(1520, 16)] * lastcol_mask)
            return carry

        lax.fori_loop(0, R, mask_last, 0)

        @pl.when(jnp.logical_and(half == 1, ch == NCHUNK - 1))
        def _():
            for k in range(64):
                rows_ref[R - 1, pl.ds(512 + 16 * k, 16)] = _ZERO16

        for g in range(GQ):
            si0 = GB * 16 * g
            accs = tuple(acc_ref[pl.ds(si0 + 16 * j, 16)] for j in range(GB))

            def row_body(r, accs, si0=si0, y0=y0):
                base = si0 + y0 + r
                return tuple(
                    accs[j] + rows_ref[r, pl.ds(base + 16 * j, 16)]
                    for j in range(GB)
                )

            accs = lax.fori_loop(0, R, row_body, accs)
            for j in range(GB):
                acc_ref[pl.ds(si0 + 16 * j, 16)] = accs[j]
        return carry

    lax.fori_loop(0, NCHUNK, chunk_body, 0)

    pltpu.sync_copy(acc_ref, shared_ref.at[s])
    plsc.subcore_barrier()

    @pl.when(s % 2 == 0)
    def _():
        pltpu.sync_copy(shared_ref.at[s + 1], tmp_ref)
        tvec = _ZERO16
        for k in range(NBLK):
            o = pl.ds(16 * k, 16)
            m = (acc_ref[o] + tmp_ref[o]) * invc_ref[o]
            obuf_ref[o] = m
            tvec = tvec + m
        mu = jnp.sum(tvec) * jnp.float32(1.0 / D)
        for k in range(NBLK):
            o = pl.ds(16 * k, 16)
            obuf_ref[o] = mu - obuf_ref[o]
        pltpu.sync_copy(obuf_ref, out_hbm.at[b])


@jax.jit
def kernel(inputs):
    invc = jnp.asarray(_INVC)
    out = _diag_mean_sc(inputs, invc)
    return out[:, :D]


# speedup vs baseline: 40.8746x; 1.2915x over previous
import functools

import numpy as np
import jax
import jax.numpy as jnp
from jax import lax
from jax.experimental import pallas as pl
from jax.experimental.pallas import tpu as pltpu
from jax.experimental.pallas import tpu_sc as plsc

B = 16
T = 1024
N = T - 1
D = T + 1
DP = 1040
R = 32
NCHUNK = 512 // R
W = 1552
NBLK = DP // 16
GQ = 5
GB = NBLK // GQ


def _inv_counts() -> np.ndarray:
    si = np.arange(DP)
    cnt = N - np.abs(si - (D // 2))
    return np.where(si < D, 1.0 / np.maximum(cnt, 1), 0.0).astype(np.float32)


_INVC = _inv_counts()

_mesh = plsc.VectorSubcoreMesh(core_axis_name="c", subcore_axis_name="s")


@functools.partial(
    pl.kernel,
    out_type=jax.ShapeDtypeStruct((B, DP), jnp.float32),
    mesh=_mesh,
    scratch_types=[
        pltpu.VMEM((R, W), jnp.float32),
        pltpu.VMEM((R, W), jnp.float32),
        pltpu.VMEM((DP,), jnp.float32),
        pltpu.VMEM((DP,), jnp.float32),
        pltpu.VMEM((DP,), jnp.float32),
        pltpu.VMEM((DP,), jnp.float32),
        pltpu.VMEM_SHARED((16, DP), jnp.float32),
        pltpu.SemaphoreType.DMA,
        pltpu.SemaphoreType.DMA,
    ],
    compiler_params=pltpu.CompilerParams(
        use_tc_tiling_on_sc=False, needs_layout_passes=False),
)
def _diag_mean_sc(in_hbm, invc_hbm, out_hbm, rows0_ref, rows1_ref, acc_ref,
                  tmp_ref, invc_ref, obuf_ref, shared_ref, sem0, sem1):
    _ZERO16 = jnp.zeros((16,), jnp.float32)
    lastcol_mask = jnp.where(
        lax.iota(jnp.int32, 16) < 15, jnp.float32(1.0), jnp.float32(0.0))
    c = lax.axis_index("c")
    s = lax.axis_index("s")
    half = s % 2
    b = c * 8 + s // 2
    y_base = half * 512
    pad = (1 - half) * 512

    def dma(ch, rbuf, sem):
        y0 = y_base + ch * R
        return pltpu.make_async_copy(
            in_hbm.at[b, pl.ds(y0, R), :],
            rbuf.at[:, pl.ds(pad, T)],
            sem,
        )

    dma(0, rows0_ref, sem0).start()

    @pl.when(half == 0)
    def _():
        def zr(r, carry):
            for k in range(32):
                rows0_ref[r, pl.ds(16 * k, 16)] = _ZERO16
                rows1_ref[r, pl.ds(16 * k, 16)] = _ZERO16
            rows0_ref[r, pl.ds(1536, 16)] = _ZERO16
            rows1_ref[r, pl.ds(1536, 16)] = _ZERO16
            return carry
        lax.fori_loop(0, R, zr, 0)

    @pl.when(half == 1)
    def _():
        def zr(r, carry):
            for k in range(33):
                rows0_ref[r, pl.ds(1024 + 16 * k, 16)] = _ZERO16
                rows1_ref[r, pl.ds(1024 + 16 * k, 16)] = _ZERO16
            return carry
        lax.fori_loop(0, R, zr, 0)

    pltpu.sync_copy(invc_hbm, invc_ref)
    for k in range(NBLK):
        acc_ref[pl.ds(16 * k, 16)] = _ZERO16

    def prep(rbuf):
        def mask_last(r, carry):
            rbuf[r, pl.ds(pad + 1008, 16)] = (
                rbuf[r, pl.ds(pad + 1008, 16)] * lastcol_mask)
            return carry
        lax.fori_loop(0, R, mask_last, 0)

    def compute(rbuf, ch):
        roff = ch * R
        for g in range(GQ):
            si0 = GB * 16 * g
            accs = tuple(acc_ref[pl.ds(si0 + 16 * j, 16)] for j in range(GB))

            def row_body(r, accs, si0=si0, roff=roff, rbuf=rbuf):
                base = si0 + roff + r
                return tuple(
                    accs[j] + rbuf[r, pl.ds(base + 16 * j, 16)]
                    for j in range(GB)
                )

            accs = lax.fori_loop(0, R, row_body, accs, unroll=4)
            for j in range(GB):
                acc_ref[pl.ds(si0 + 16 * j, 16)] = accs[j]

    def loop_i(i, carry):
        ch0 = 2 * i
        ch1 = 2 * i + 1
        dma(ch1, rows1_ref, sem1).start()
        dma(ch0, rows0_ref, sem0).wait()
        prep(rows0_ref)
        compute(rows0_ref, ch0)

        @pl.when(i < NCHUNK // 2 - 1)
        def _():
            dma(ch0 + 2, rows0_ref, sem0).start()

        dma(ch1, rows1_ref, sem1).wait()
        prep(rows1_ref)

        @pl.when(jnp.logical_and(half == 1, ch1 == NCHUNK - 1))
        def _():
            for k in range(64):
                rows1_ref[R - 1, pl.ds(16 * k, 16)] = _ZERO16

        compute(rows1_ref, ch1)
        return carry

    lax.fori_loop(0, NCHUNK // 2, loop_i, 0)

    pltpu.sync_copy(acc_ref, shared_ref.at[s])
    plsc.subcore_barrier()

    @pl.when(s % 2 == 0)
    def _():
        pltpu.sync_copy(shared_ref.at[s + 1], tmp_ref)
        tvec = _ZERO16
        for k in range(NBLK):
            o = pl.ds(16 * k, 16)
            m = (acc_ref[o] + tmp_ref[o]) * invc_ref[o]
            obuf_ref[o] = m
            tvec = tvec + m
        mu = jnp.sum(tvec) * jnp.float32(1.0 / D)
        for k in range(NBLK):
            o = pl.ds(16 * k, 16)
            obuf_ref[o] = mu - obuf_ref[o]
        pltpu.sync_copy(obuf_ref, out_hbm.at[b])


@jax.jit
def kernel(inputs):
    invc = jnp.asarray(_INVC)
    out = _diag_mean_sc(inputs, invc)
    return out[:, :D]
